# baseline (speedup 1.0000x reference)
import functools

import jax
import jax.numpy as jnp
from jax import lax
from jax.experimental import pallas as pl
from jax.experimental.pallas import tpu as pltpu
from jax.experimental.pallas import tpu_sc as plsc

N_D = 64
LANES = 16
HCH = 4


def _sc_embed(w_flat, idx_t, hist, n_batch):
    info = plsc.get_sparse_core_info()
    num_workers = info.num_cores * info.num_subcores
    bpw = n_batch // num_workers
    n_chunks = hist // HCH
    nc2 = n_chunks // 2
    nvec = bpw // LANES
    mesh = plsc.VectorSubcoreMesh(core_axis_name="c", subcore_axis_name="s")

    @functools.partial(
        pl.kernel,
        mesh=mesh,
        out_type=jax.ShapeDtypeStruct((hist, N_D, n_batch), jnp.float32),
        scratch_types=[
            pltpu.VMEM((2 * N_D,), jnp.float32),
            pltpu.VMEM((hist, bpw), jnp.int32),
            pltpu.VMEM((HCH, N_D, bpw), jnp.float32),
            pltpu.VMEM((HCH, N_D, bpw), jnp.float32),
            pltpu.SemaphoreType.DMA,
            pltpu.SemaphoreType.DMA,
        ],
    )
    def k(w_hbm, idx_hbm, out_hbm, w_v, idx_v, rows0, rows1, semo0, semo1):
        wid = lax.axis_index("s") * info.num_cores + lax.axis_index("c")
        b0 = pl.multiple_of(wid * bpw, bpw)

        pltpu.sync_copy(idx_hbm.at[pl.ds(0, hist), pl.ds(b0, bpw)], idx_v)
        pltpu.sync_copy(w_hbm, w_v)
        wv0 = [w_v[pl.ds(j * LANES, LANES)] for j in range(N_D // LANES)]
        wv1 = [w_v[pl.ds(N_D + j * LANES, LANES)] for j in range(N_D // LANES)]
        w0s = [wv0[j][l] for j in range(N_D // LANES) for l in range(LANES)]
        difs = [wv1[j][l] - w0s[j * LANES + l]
                for j in range(N_D // LANES) for l in range(LANES)]

        def compute(i, rows_v):
            for hh in range(HCH):
                h = i * HCH + hh
                fv = [idx_v[h, pl.ds(c * LANES, LANES)].astype(jnp.float32)
                      for c in range(nvec)]
                for d in range(N_D):
                    for c in range(nvec):
                        rows_v[hh, d, pl.ds(c * LANES, LANES)] = (
                            w0s[d] + fv[c] * difs[d])

        bufs = ((rows0, semo0), (rows1, semo1))

        def outer(i2, carry):
            for b, (rows_v, semo) in enumerate(bufs):
                i = 2 * i2 + b
                hoff = pl.multiple_of(i * HCH, HCH)

                @pl.when(i2 > 0)
                def _drain():
                    pltpu.make_async_copy(
                        rows_v,
                        out_hbm.at[pl.ds(hoff, HCH), pl.ds(0, N_D),
                                   pl.ds(b0, bpw)],
                        semo).wait()

                compute(i, rows_v)
                pltpu.async_copy(
                    rows_v,
                    out_hbm.at[pl.ds(hoff, HCH), pl.ds(0, N_D),
                               pl.ds(b0, bpw)],
                    semo)
            return carry

        lax.fori_loop(0, nc2, outer, 0)

        for b, (rows_v, semo) in enumerate(bufs):
            hoff = pl.multiple_of((n_chunks - 2 + b) * HCH, HCH)
            pltpu.make_async_copy(
                rows_v,
                out_hbm.at[pl.ds(hoff, HCH), pl.ds(0, N_D), pl.ds(b0, bpw)],
                semo).wait()

    return k(w_flat, idx_t)


def kernel(input, weight):
    b, h = input.shape
    out_t = _sc_embed(weight.reshape(2 * N_D), input.T, h, b)
    return jnp.transpose(out_t, (2, 0, 1))


# speedup vs baseline: 2.1628x; 2.1628x over previous
import functools

import jax
import jax.numpy as jnp
from jax import lax
from jax.experimental import pallas as pl
from jax.experimental.pallas import tpu as pltpu
from jax.experimental.pallas import tpu_sc as plsc

N_D = 64
LANES = 16
HCH = 5


def _sc_embed(w_flat, idx_t, hist, n_batch):
    info = plsc.get_sparse_core_info()
    num_workers = info.num_cores * info.num_subcores
    bpw = n_batch // num_workers
    n_chunks = hist // HCH
    nc2 = n_chunks // 2
    nvec = bpw // LANES
    mesh = plsc.VectorSubcoreMesh(core_axis_name="c", subcore_axis_name="s")

    @functools.partial(
        pl.kernel,
        mesh=mesh,
        out_type=jax.ShapeDtypeStruct((hist, N_D, n_batch), jnp.float32),
        scratch_types=[
            pltpu.VMEM((2 * N_D,), jnp.float32),
            pltpu.VMEM((hist, bpw), jnp.int32),
            pltpu.VMEM((HCH, N_D, bpw), jnp.float32),
            pltpu.VMEM((HCH, N_D, bpw), jnp.float32),
            pltpu.SemaphoreType.DMA,
            pltpu.SemaphoreType.DMA,
        ],
    )
    def k(w_hbm, idx_hbm, out_hbm, w_v, idx_v, rows0, rows1, semo0, semo1):
        wid = lax.axis_index("s") * info.num_cores + lax.axis_index("c")
        b0 = pl.multiple_of(wid * bpw, bpw)

        pltpu.sync_copy(idx_hbm.at[pl.ds(0, hist), pl.ds(b0, bpw)], idx_v)
        pltpu.sync_copy(w_hbm, w_v)
        wv0 = [w_v[pl.ds(j * LANES, LANES)] for j in range(N_D // LANES)]
        wv1 = [w_v[pl.ds(N_D + j * LANES, LANES)] for j in range(N_D // LANES)]
        w0s = [wv0[j][l] for j in range(N_D // LANES) for l in range(LANES)]
        difs = [wv1[j][l] - w0s[j * LANES + l]
                for j in range(N_D // LANES) for l in range(LANES)]

        def compute(i, rows_v):
            def h_body(hh, carry):
                h = i * HCH + hh
                fv = [idx_v[h, pl.ds(c * LANES, LANES)].astype(jnp.float32)
                      for c in range(nvec)]
                for d in range(N_D):
                    for c in range(nvec):
                        rows_v[hh, d, pl.ds(c * LANES, LANES)] = (
                            w0s[d] + fv[c] * difs[d])
                return carry
            lax.fori_loop(0, HCH, h_body, 0)

        bufs = ((rows0, semo0), (rows1, semo1))

        def outer(i2, carry):
            for b, (rows_v, semo) in enumerate(bufs):
                i = 2 * i2 + b
                hoff = pl.multiple_of(i * HCH, HCH)

                @pl.when(i2 > 0)
                def _drain():
                    pltpu.make_async_copy(
                        rows_v,
                        out_hbm.at[pl.ds(hoff, HCH), pl.ds(0, N_D),
                                   pl.ds(b0, bpw)],
                        semo).wait()

                compute(i, rows_v)
                pltpu.async_copy(
                    rows_v,
                    out_hbm.at[pl.ds(hoff, HCH), pl.ds(0, N_D),
                               pl.ds(b0, bpw)],
                    semo)
            return carry

        lax.fori_loop(0, nc2, outer, 0)

        for b, (rows_v, semo) in enumerate(bufs):
            hoff = pl.multiple_of((n_chunks - 2 + b) * HCH, HCH)
            pltpu.make_async_copy(
                rows_v,
                out_hbm.at[pl.ds(hoff, HCH), pl.ds(0, N_D), pl.ds(b0, bpw)],
                semo).wait()

    return k(w_flat, idx_t)


def kernel(input, weight):
    b, h = input.shape
    out_t = _sc_embed(weight.reshape(2 * N_D), input.T, h, b)
    return jnp.transpose(out_t, (2, 0, 1))


# speedup vs baseline: 2.3792x; 1.1001x over previous
import functools

import jax
import jax.numpy as jnp
from jax import lax
from jax.experimental import pallas as pl
from jax.experimental.pallas import tpu as pltpu
from jax.experimental.pallas import tpu_sc as plsc

N_D = 64
LANES = 16
HCH = 4


def _sc_embed(w_flat, idx_t, hist, n_batch):
    info = plsc.get_sparse_core_info()
    num_workers = info.num_cores * info.num_subcores
    bpw = n_batch // num_workers
    n_chunks = hist // HCH
    nc2 = n_chunks // 2
    nvec = bpw // LANES
    mesh = plsc.VectorSubcoreMesh(core_axis_name="c", subcore_axis_name="s")

    @functools.partial(
        pl.kernel,
        mesh=mesh,
        out_type=jax.ShapeDtypeStruct((hist, N_D, n_batch), jnp.float32),
        scratch_types=[
            pltpu.VMEM((2 * N_D,), jnp.float32),
            pltpu.VMEM((hist, bpw), jnp.int32),
            pltpu.VMEM((HCH, N_D, bpw), jnp.float32),
            pltpu.VMEM((HCH, N_D, bpw), jnp.float32),
            pltpu.SemaphoreType.DMA,
            pltpu.SemaphoreType.DMA,
        ],
    )
    def k(w_hbm, idx_hbm, out_hbm, w_v, idx_v, rows0, rows1, semo0, semo1):
        wid = lax.axis_index("s") * info.num_cores + lax.axis_index("c")
        b0 = pl.multiple_of(wid * bpw, bpw)

        pltpu.sync_copy(idx_hbm.at[pl.ds(0, hist), pl.ds(b0, bpw)], idx_v)
        pltpu.sync_copy(w_hbm, w_v)
        wv0 = [w_v[pl.ds(j * LANES, LANES)] for j in range(N_D // LANES)]
        wv1 = [w_v[pl.ds(N_D + j * LANES, LANES)] for j in range(N_D // LANES)]
        w0s = [wv0[j][l] for j in range(N_D // LANES) for l in range(LANES)]
        difs = [wv1[j][l] - w0s[j * LANES + l]
                for j in range(N_D // LANES) for l in range(LANES)]

        def compute(i, rows_v):
            def h_body(hh, carry):
                h = i * HCH + hh
                fv = [idx_v[h, pl.ds(c * LANES, LANES)].astype(jnp.float32)
                      for c in range(nvec)]
                for d in range(N_D):
                    for c in range(nvec):
                        rows_v[hh, d, pl.ds(c * LANES, LANES)] = (
                            w0s[d] + fv[c] * difs[d])
                return carry
            lax.fori_loop(0, HCH, h_body, 0)

        bufs = ((rows0, semo0), (rows1, semo1))

        def outer(i2, carry):
            for b, (rows_v, semo) in enumerate(bufs):
                i = 2 * i2 + b
                hoff = pl.multiple_of(i * HCH, HCH)

                @pl.when(i2 > 0)
                def _drain():
                    pltpu.make_async_copy(
                        rows_v,
                        out_hbm.at[pl.ds(hoff, HCH), pl.ds(0, N_D),
                                   pl.ds(b0, bpw)],
                        semo).wait()

                compute(i, rows_v)
                pltpu.async_copy(
                    rows_v,
                    out_hbm.at[pl.ds(hoff, HCH), pl.ds(0, N_D),
                               pl.ds(b0, bpw)],
                    semo)
            return carry

        lax.fori_loop(0, nc2, outer, 0)

        for b, (rows_v, semo) in enumerate(bufs):
            hoff = pl.multiple_of((n_chunks - 2 + b) * HCH, HCH)
            pltpu.make_async_copy(
                rows_v,
                out_hbm.at[pl.ds(hoff, HCH), pl.ds(0, N_D), pl.ds(b0, bpw)],
                semo).wait()

    return k(w_flat, idx_t)


def kernel(input, weight):
    b, h = input.shape
    out_t = _sc_embed(weight.reshape(2 * N_D), input.T, h, b)
    return jnp.transpose(out_t, (2, 0, 1))
